# baseline (speedup 1.0000x reference)
import functools

import jax
import jax.numpy as jnp
from jax import lax
from jax.experimental import pallas as pl
from jax.experimental.pallas import tpu as pltpu
from jax.experimental.pallas import tpu_sc as plsc

N_PAD = 10240
D = 256
TPW = 320
E_PAD = 163840
SCHUNK = 4096
NSTAGE = E_PAD // SCHUNK
LCAP = 6144
LCAPC = LCAP // 128
CHUNK = 128

_f32 = jnp.float32
_i32 = jnp.int32


def _route_body(src_hbm, dst_hbm, srcl_hbm, dstl_hbm, cnts_hbm, degp_hbm,
                sst, dstt, sst_b, dstt_b, slst, dlst, dega, cbuf,
                sems0, sems1):
    c = lax.axis_index("c")
    s = lax.axis_index("s")
    w = s * 2 + c
    lo = w * TPW
    zero16 = jnp.zeros((16,), _f32)
    one16 = jnp.ones((16,), _f32)

    def degz(r, carry):
        dega[pl.ds(r * 16, 16)] = zero16
        return carry
    lax.fori_loop(0, TPW + 1, degz, 0)

    iota16 = lax.iota(_i32, 16)

    def block2(qq, cntv, bs, bd):
        for k in range(8):
            dv = bd[pl.ds(qq * 128 + k * 16, 16)]
            sv = bs[pl.ds(qq * 128 + k * 16, 16)]
            lv = dv - lo
            m = jnp.logical_and(lv >= 0, lv < TPW)
            cs = plsc.cumsum(m.astype(_i32))
            pos = jnp.where(m, cntv + cs - 1, LCAP - 1)
            plsc.store_scatter(slst, [pos], sv)
            plsc.store_scatter(dlst, [pos], lv)
            cntv = cntv + plsc.all_reduce_population_count(m)
        return cntv

    def block(qq, cntv):
        return block2(qq, cntv, sst, dstt)

    def sstart(t, sb_s, sb_d, sem):
        pltpu.async_copy(src_hbm.at[pl.ds(t * SCHUNK, SCHUNK)], sb_s, sem)
        pltpu.async_copy(dst_hbm.at[pl.ds(t * SCHUNK, SCHUNK)], sb_d, sem)

    def swait(sb_s, sb_d, sem):
        pltpu.make_async_copy(src_hbm.at[pl.ds(0, SCHUNK)], sb_s, sem).wait()
        pltpu.make_async_copy(src_hbm.at[pl.ds(0, SCHUNK)], sb_d, sem).wait()

    sstart(0, sst, dstt, sems0)

    def stage2(i, cntv):
        t = i * 2

        @pl.when(t + 1 < NSTAGE)
        def _p1():
            sstart(t + 1, sst_b, dstt_b, sems1)
        swait(sst, dstt, sems0)
        cntv = lax.fori_loop(0, SCHUNK // 128, block, cntv)

        @pl.when(t + 2 < NSTAGE)
        def _p2():
            sstart(t + 2, sst, dstt, sems0)
        swait(sst_b, dstt_b, sems1)

        def block_b(qq, cv):
            return block2(qq, cv, sst_b, dstt_b)
        return lax.fori_loop(0, SCHUNK // 128, block_b, cntv)

    cntv = lax.fori_loop(0, NSTAGE // 2, stage2, jnp.zeros((16,), _i32))
    cnt = cntv[0]

    zpad = jnp.zeros((16,), _i32)
    dpad = jnp.full((16,), TPW, _i32)
    for k in range(CHUNK // 16):
        ppos = cnt + k * 16 + iota16
        plsc.store_scatter(slst, [ppos], zpad)
        plsc.store_scatter(dlst, [ppos], dpad)

    def deg_group(q, carry):
        d16 = dlst[pl.ds(q * 16, 16)]
        locs = [d16[e] for e in range(16)]
        for e in range(16):
            plsc.addupdate(dega.at[pl.ds(locs[e] * 16, 16)], one16)
        return carry
    lax.fori_loop(0, (cnt + 15) // 16, deg_group, 0)

    pltpu.sync_copy(slst, srcl_hbm.at[w])
    pltpu.sync_copy(dlst, dstl_hbm.at[w])
    cbuf[pl.ds(0, 16)] = cntv
    pltpu.sync_copy(cbuf, cnts_hbm.at[w])
    pltpu.sync_copy(dega, degp_hbm.at[w])


@functools.lru_cache(maxsize=None)
def _route():
    return pl.kernel(
        _route_body,
        out_type=(jax.ShapeDtypeStruct((32, LCAP), _i32),
                  jax.ShapeDtypeStruct((32, LCAP), _i32),
                  jax.ShapeDtypeStruct((32, 16), _i32),
                  jax.ShapeDtypeStruct((32, (TPW + 1) * 16), _f32)),
        mesh=plsc.VectorSubcoreMesh(core_axis_name="c", subcore_axis_name="s",
                                    num_cores=2, num_subcores=16),
        scratch_types=[
            pltpu.VMEM((SCHUNK,), _i32),
            pltpu.VMEM((SCHUNK,), _i32),
            pltpu.VMEM((SCHUNK,), _i32),
            pltpu.VMEM((SCHUNK,), _i32),
            pltpu.VMEM((LCAP,), _i32),
            pltpu.VMEM((LCAP,), _i32),
            pltpu.VMEM(((TPW + 1) * 16,), _f32),
            pltpu.VMEM((16,), _i32),
            pltpu.SemaphoreType.DMA,
            pltpu.SemaphoreType.DMA,
        ],
        compiler_params=pltpu.CompilerParams(needs_layout_passes=False),
    )


GCH = 64


def _hop_body(srcl_hbm, dstl_hbm, cnts_hbm, g_hbm, out_hbm,
              slst2, cbuf, dl0, dl1, rows0, rows1, acc,
              sg0, sg1, sd0, sd1):
    c = lax.axis_index("c")
    s = lax.axis_index("s")
    w = s * 2 + c
    pltpu.sync_copy(srcl_hbm.at[w], slst2)
    pltpu.sync_copy(cnts_hbm.at[w], cbuf)
    cnt = cbuf[pl.ds(0, 16)][0]
    pltpu.sync_copy(g_hbm.at[pl.ds(w * TPW, TPW)], acc.at[pl.ds(0, TPW)])

    trips = (cnt + GCH - 1) // GCH

    def start(q, rows, dl, sg, sd):
        pltpu.async_copy(g_hbm.at[slst2.at[q]], rows, sg)
        pltpu.async_copy(dstl_hbm.at[w, q], dl, sd)

    def wait(rows, dl, sg, sd):
        pltpu.make_async_copy(g_hbm.at[pl.ds(0, GCH)], rows, sg).wait()
        pltpu.make_async_copy(dstl_hbm.at[0, 0], dl, sd).wait()

    def adds(dl, rows):
        vecs = [dl[pl.ds(u * 16, 16)] for u in range(GCH // 16)]
        locs = [v[e] for v in vecs for e in range(16)]
        for e in range(GCH):
            for f in range(D // 16):
                plsc.addupdate(acc.at[locs[e], pl.ds(f * 16, 16)],
                               rows[e, pl.ds(f * 16, 16)])

    @pl.when(trips > 0)
    def _prologue():
        start(0, rows0, dl0, sg0, sd0)

    def chunk2(i, carry):
        q = i * 2

        @pl.when(q + 1 < trips)
        def _pref1():
            start(q + 1, rows1, dl1, sg1, sd1)
        wait(rows0, dl0, sg0, sd0)
        adds(dl0, rows0)

        @pl.when(q + 2 < trips)
        def _pref2():
            start(q + 2, rows0, dl0, sg0, sd0)

        @pl.when(q + 1 < trips)
        def _do1():
            wait(rows1, dl1, sg1, sd1)
            adds(dl1, rows1)
        return carry

    lax.fori_loop(0, (trips + 1) // 2, chunk2, 0)
    pltpu.sync_copy(acc.at[pl.ds(0, TPW)], out_hbm.at[pl.ds(w * TPW, TPW)])


@functools.lru_cache(maxsize=None)
def _hop():
    return pl.kernel(
        _hop_body,
        out_type=jax.ShapeDtypeStruct((N_PAD, D), _f32),
        mesh=plsc.VectorSubcoreMesh(core_axis_name="c", subcore_axis_name="s",
                                    num_cores=2, num_subcores=16),
        scratch_types=[
            pltpu.VMEM((LCAP // GCH, GCH), _i32),
            pltpu.VMEM((16,), _i32),
            pltpu.VMEM((GCH,), _i32),
            pltpu.VMEM((GCH,), _i32),
            pltpu.VMEM((GCH, D), _f32),
            pltpu.VMEM((GCH, D), _f32),
            pltpu.VMEM((TPW + 1, D), _f32),
            pltpu.SemaphoreType.DMA,
            pltpu.SemaphoreType.DMA,
            pltpu.SemaphoreType.DMA,
            pltpu.SemaphoreType.DMA,
        ],
    )


def _prep_body(deg_ref, x_ref, g0_ref, d_ref, d2_ref):
    deg = deg_ref[...] + 1.0
    d = lax.rsqrt(deg)
    d_ref[...] = d
    d2_ref[...] = 1.0 / deg
    g0_ref[...] = x_ref[...] * d


def _mid_body(s1_ref, d2_ref, g1_ref):
    g1_ref[...] = s1_ref[...] * d2_ref[...]


def _mm_body(s2_ref, d_ref, w_ref, b_ref, o_ref):
    xs = s2_ref[...] * d_ref[...]
    o_ref[...] = lax.dot_general(
        xs, w_ref[...], (((1,), (1,)), ((), ())),
        preferred_element_type=_f32) + b_ref[...]


def kernel(V, E, X, W, b):
    del V
    n = X.shape[0]
    e = E.shape[1]
    src = E[0].astype(_i32)
    dst = E[1].astype(_i32)
    src_p = jnp.concatenate([src, jnp.zeros((E_PAD - e,), _i32)])
    dst_p = jnp.concatenate([dst, jnp.full((E_PAD - e,), -1, _i32)])
    x_p = jnp.pad(X, ((0, N_PAD - n), (0, 0)))

    srcl, dstl, cnts, degp = _route()(src_p, dst_p)
    srcl3 = srcl.reshape(32, LCAP // GCH, GCH)
    dstl3 = dstl.reshape(32, LCAP // GCH, GCH)
    deg = degp.reshape(32, TPW + 1, 16)[:, :TPW, 0].reshape(N_PAD, 1)

    hop = _hop()

    rb = 2048
    grid = (N_PAD // rb,)
    blkD = pl.BlockSpec((rb, D), lambda i: (i, 0))
    blk1 = pl.BlockSpec((rb, 1), lambda i: (i, 0))
    g0, d, d2 = pl.pallas_call(
        _prep_body,
        grid=grid,
        in_specs=[blk1, blkD],
        out_specs=[blkD, blk1, blk1],
        out_shape=[jax.ShapeDtypeStruct((N_PAD, D), _f32),
                   jax.ShapeDtypeStruct((N_PAD, 1), _f32),
                   jax.ShapeDtypeStruct((N_PAD, 1), _f32)],
    )(deg, x_p)

    s1 = hop(srcl3, dstl3, cnts, g0)
    g1 = pl.pallas_call(
        _mid_body,
        grid=grid,
        in_specs=[blkD, blk1],
        out_specs=blkD,
        out_shape=jax.ShapeDtypeStruct((N_PAD, D), _f32),
    )(s1, d2)

    s2 = hop(srcl3, dstl3, cnts, g1)

    mb = 2000
    out = pl.pallas_call(
        _mm_body,
        grid=(n // mb,),
        in_specs=[pl.BlockSpec((mb, D), lambda i: (i, 0)),
                  pl.BlockSpec((mb, 1), lambda i: (i, 0)),
                  pl.BlockSpec((D, D), lambda i: (0, 0)),
                  pl.BlockSpec((1, D), lambda i: (0, 0))],
        out_specs=pl.BlockSpec((mb, D), lambda i: (i, 0)),
        out_shape=jax.ShapeDtypeStruct((n, D), _f32),
    )(s2[:n], d[:n], W, b.reshape(1, D))
    return out


# speedup vs baseline: 1.1879x; 1.1879x over previous
import functools

import jax
import jax.numpy as jnp
from jax import lax
from jax.experimental import pallas as pl
from jax.experimental.pallas import tpu as pltpu
from jax.experimental.pallas import tpu_sc as plsc

N_PAD = 10240
D = 256
TPW = 320
E_PAD = 163840
SCHUNK = 4096
NSTAGE = E_PAD // SCHUNK
LCAP = 6144
LCAPC = LCAP // 128
CHUNK = 128

_f32 = jnp.float32
_i32 = jnp.int32


def _route_body(src_hbm, dst_hbm, srcl_hbm, dstl_hbm, cnts_hbm, degp_hbm,
                sst, dstt, sst_b, dstt_b, slst, dlst, dega, cbuf,
                sems0, sems1):
    c = lax.axis_index("c")
    s = lax.axis_index("s")
    w = s * 2 + c
    lo = w * TPW
    zero16 = jnp.zeros((16,), _f32)
    one16 = jnp.ones((16,), _f32)

    def degz(r, carry):
        dega[pl.ds(r * 16, 16)] = zero16
        return carry
    lax.fori_loop(0, TPW + 1, degz, 0)

    iota16 = lax.iota(_i32, 16)

    def block2(qq, cntv, bs, bd):
        for k in range(8):
            dv = bd[pl.ds(qq * 128 + k * 16, 16)]
            sv = bs[pl.ds(qq * 128 + k * 16, 16)]
            lv = dv - lo
            m = jnp.logical_and(lv >= 0, lv < TPW)
            cs = plsc.cumsum(m.astype(_i32))
            pos = jnp.where(m, cntv + cs - 1, LCAP - 1)
            plsc.store_scatter(slst, [pos], sv)
            plsc.store_scatter(dlst, [pos], lv)
            cntv = cntv + plsc.all_reduce_population_count(m)
        return cntv

    def block(qq, cntv):
        return block2(qq, cntv, sst, dstt)

    def sstart(t, sb_s, sb_d, sem):
        pltpu.async_copy(src_hbm.at[pl.ds(t * SCHUNK, SCHUNK)], sb_s, sem)
        pltpu.async_copy(dst_hbm.at[pl.ds(t * SCHUNK, SCHUNK)], sb_d, sem)

    def swait(sb_s, sb_d, sem):
        pltpu.make_async_copy(src_hbm.at[pl.ds(0, SCHUNK)], sb_s, sem).wait()
        pltpu.make_async_copy(src_hbm.at[pl.ds(0, SCHUNK)], sb_d, sem).wait()

    sstart(0, sst, dstt, sems0)

    def stage2(i, cntv):
        t = i * 2

        @pl.when(t + 1 < NSTAGE)
        def _p1():
            sstart(t + 1, sst_b, dstt_b, sems1)
        swait(sst, dstt, sems0)
        cntv = lax.fori_loop(0, SCHUNK // 128, block, cntv)

        @pl.when(t + 2 < NSTAGE)
        def _p2():
            sstart(t + 2, sst, dstt, sems0)
        swait(sst_b, dstt_b, sems1)

        def block_b(qq, cv):
            return block2(qq, cv, sst_b, dstt_b)
        return lax.fori_loop(0, SCHUNK // 128, block_b, cntv)

    cntv = lax.fori_loop(0, NSTAGE // 2, stage2, jnp.zeros((16,), _i32))
    cnt = cntv[0]

    zpad = jnp.zeros((16,), _i32)
    dpad = jnp.full((16,), TPW, _i32)
    for k in range(CHUNK // 16):
        ppos = cnt + k * 16 + iota16
        plsc.store_scatter(slst, [ppos], zpad)
        plsc.store_scatter(dlst, [ppos], dpad)

    def deg_group(q, carry):
        d16 = dlst[pl.ds(q * 16, 16)]
        locs = [d16[e] for e in range(16)]
        for e in range(16):
            plsc.addupdate(dega.at[pl.ds(locs[e] * 16, 16)], one16)
        return carry
    lax.fori_loop(0, (cnt + 15) // 16, deg_group, 0)

    pltpu.sync_copy(slst, srcl_hbm.at[w])
    pltpu.sync_copy(dlst, dstl_hbm.at[w])
    cbuf[pl.ds(0, 16)] = cntv
    pltpu.sync_copy(cbuf, cnts_hbm.at[w])
    pltpu.sync_copy(dega, degp_hbm.at[w])


@functools.lru_cache(maxsize=None)
def _route():
    return pl.kernel(
        _route_body,
        out_type=(jax.ShapeDtypeStruct((32, LCAP), _i32),
                  jax.ShapeDtypeStruct((32, LCAP), _i32),
                  jax.ShapeDtypeStruct((32, 16), _i32),
                  jax.ShapeDtypeStruct((32, (TPW + 1) * 16), _f32)),
        mesh=plsc.VectorSubcoreMesh(core_axis_name="c", subcore_axis_name="s",
                                    num_cores=2, num_subcores=16),
        scratch_types=[
            pltpu.VMEM((SCHUNK,), _i32),
            pltpu.VMEM((SCHUNK,), _i32),
            pltpu.VMEM((SCHUNK,), _i32),
            pltpu.VMEM((SCHUNK,), _i32),
            pltpu.VMEM((LCAP,), _i32),
            pltpu.VMEM((LCAP,), _i32),
            pltpu.VMEM(((TPW + 1) * 16,), _f32),
            pltpu.VMEM((16,), _i32),
            pltpu.SemaphoreType.DMA,
            pltpu.SemaphoreType.DMA,
        ],
        compiler_params=pltpu.CompilerParams(needs_layout_passes=False),
    )


GCH = 64


def _hop_body(srcl_hbm, dstl_hbm, cnts_hbm, g_hbm, out_hbm,
              slst2, cbuf, dl0, dl1, rows0, rows1, acc,
              sg0, sg1, sd0, sd1):
    c = lax.axis_index("c")
    s = lax.axis_index("s")
    w = s * 2 + c
    pltpu.sync_copy(srcl_hbm.at[w], slst2)
    pltpu.sync_copy(cnts_hbm.at[w], cbuf)
    cnt = cbuf[pl.ds(0, 16)][0]
    pltpu.sync_copy(g_hbm.at[pl.ds(w * TPW, TPW)], acc.at[pl.ds(0, TPW)])

    trips = (cnt + GCH - 1) // GCH

    def start(q, rows, dl, sg, sd):
        pltpu.async_copy(g_hbm.at[slst2.at[q]], rows, sg)
        pltpu.async_copy(dstl_hbm.at[w, q], dl, sd)

    def wait(rows, dl, sg, sd):
        pltpu.make_async_copy(g_hbm.at[pl.ds(0, GCH)], rows, sg).wait()
        pltpu.make_async_copy(dstl_hbm.at[0, 0], dl, sd).wait()

    def adds(dl, rows):
        def group(u, carry2):
            d16 = dl[pl.ds(u * 16, 16)]
            locs = [d16[e] for e in range(16)]
            for e in range(16):
                for f in range(D // 16):
                    plsc.addupdate(acc.at[locs[e], pl.ds(f * 16, 16)],
                                   rows[u * 16 + e, pl.ds(f * 16, 16)])
            return carry2
        lax.fori_loop(0, GCH // 16, group, 0)

    @pl.when(trips > 0)
    def _prologue():
        start(0, rows0, dl0, sg0, sd0)

    def chunk2(i, carry):
        q = i * 2

        @pl.when(q + 1 < trips)
        def _pref1():
            start(q + 1, rows1, dl1, sg1, sd1)
        wait(rows0, dl0, sg0, sd0)
        adds(dl0, rows0)

        @pl.when(q + 2 < trips)
        def _pref2():
            start(q + 2, rows0, dl0, sg0, sd0)

        @pl.when(q + 1 < trips)
        def _do1():
            wait(rows1, dl1, sg1, sd1)
            adds(dl1, rows1)
        return carry

    lax.fori_loop(0, (trips + 1) // 2, chunk2, 0)
    pltpu.sync_copy(acc.at[pl.ds(0, TPW)], out_hbm.at[pl.ds(w * TPW, TPW)])


@functools.lru_cache(maxsize=None)
def _hop():
    return pl.kernel(
        _hop_body,
        out_type=jax.ShapeDtypeStruct((N_PAD, D), _f32),
        mesh=plsc.VectorSubcoreMesh(core_axis_name="c", subcore_axis_name="s",
                                    num_cores=2, num_subcores=16),
        scratch_types=[
            pltpu.VMEM((LCAP // GCH, GCH), _i32),
            pltpu.VMEM((16,), _i32),
            pltpu.VMEM((GCH,), _i32),
            pltpu.VMEM((GCH,), _i32),
            pltpu.VMEM((GCH, D), _f32),
            pltpu.VMEM((GCH, D), _f32),
            pltpu.VMEM((TPW + 1, D), _f32),
            pltpu.SemaphoreType.DMA,
            pltpu.SemaphoreType.DMA,
            pltpu.SemaphoreType.DMA,
            pltpu.SemaphoreType.DMA,
        ],
    )


def _prep_body(deg_ref, x_ref, g0_ref, d_ref, d2_ref):
    deg = deg_ref[...] + 1.0
    d = lax.rsqrt(deg)
    d_ref[...] = d
    d2_ref[...] = 1.0 / deg
    g0_ref[...] = x_ref[...] * d


def _mid_body(s1_ref, d2_ref, g1_ref):
    g1_ref[...] = s1_ref[...] * d2_ref[...]


def _mm_body(s2_ref, d_ref, w_ref, b_ref, o_ref):
    xs = s2_ref[...] * d_ref[...]
    o_ref[...] = lax.dot_general(
        xs, w_ref[...], (((1,), (1,)), ((), ())),
        preferred_element_type=_f32) + b_ref[...]


def kernel(V, E, X, W, b):
    del V
    n = X.shape[0]
    e = E.shape[1]
    src = E[0].astype(_i32)
    dst = E[1].astype(_i32)
    src_p = jnp.concatenate([src, jnp.zeros((E_PAD - e,), _i32)])
    dst_p = jnp.concatenate([dst, jnp.full((E_PAD - e,), -1, _i32)])
    x_p = jnp.pad(X, ((0, N_PAD - n), (0, 0)))

    srcl, dstl, cnts, degp = _route()(src_p, dst_p)
    srcl3 = srcl.reshape(32, LCAP // GCH, GCH)
    dstl3 = dstl.reshape(32, LCAP // GCH, GCH)
    deg = degp.reshape(32, TPW + 1, 16)[:, :TPW, 0].reshape(N_PAD, 1)

    hop = _hop()

    rb = 2048
    grid = (N_PAD // rb,)
    blkD = pl.BlockSpec((rb, D), lambda i: (i, 0))
    blk1 = pl.BlockSpec((rb, 1), lambda i: (i, 0))
    g0, d, d2 = pl.pallas_call(
        _prep_body,
        grid=grid,
        in_specs=[blk1, blkD],
        out_specs=[blkD, blk1, blk1],
        out_shape=[jax.ShapeDtypeStruct((N_PAD, D), _f32),
                   jax.ShapeDtypeStruct((N_PAD, 1), _f32),
                   jax.ShapeDtypeStruct((N_PAD, 1), _f32)],
    )(deg, x_p)

    s1 = hop(srcl3, dstl3, cnts, g0)
    g1 = pl.pallas_call(
        _mid_body,
        grid=grid,
        in_specs=[blkD, blk1],
        out_specs=blkD,
        out_shape=jax.ShapeDtypeStruct((N_PAD, D), _f32),
    )(s1, d2)

    s2 = hop(srcl3, dstl3, cnts, g1)

    mb = 2000
    out = pl.pallas_call(
        _mm_body,
        grid=(n // mb,),
        in_specs=[pl.BlockSpec((mb, D), lambda i: (i, 0)),
                  pl.BlockSpec((mb, 1), lambda i: (i, 0)),
                  pl.BlockSpec((D, D), lambda i: (0, 0)),
                  pl.BlockSpec((1, D), lambda i: (0, 0))],
        out_specs=pl.BlockSpec((mb, D), lambda i: (i, 0)),
        out_shape=jax.ShapeDtypeStruct((n, D), _f32),
    )(s2[:n], d[:n], W, b.reshape(1, D))
    return out
